# baseline (speedup 1.0000x reference)
import jax
import jax.numpy as jnp
from jax import lax
from jax.experimental import pallas as pl
from jax.experimental.pallas import tpu as pltpu
from jax.experimental.pallas import tpu_sc as plsc

B = 16384
L = 200
V = 256
D = 128

NUM_CORES = 2
NUM_SUBCORES = 16
NW = NUM_CORES * NUM_SUBCORES
LANES = 16

ROWS_PER_W = B // NW
CHUNK = 64
N_CHUNKS = ROWS_PER_W // CHUNK
GROUPS = CHUNK // LANES

PV = L // 2
PP = 128


def _hist_body(ids_hbm, lo_hbm, hi_hbm, ids_v0, ids_v1, counts_v0, counts_v1,
               sem_in0, sem_in1, sem_out0, sem_out1):
  wid = lax.axis_index("s") * NUM_CORES + lax.axis_index("c")
  lane = lax.iota(jnp.int32, LANES)
  ones = jnp.ones((LANES,), jnp.float32)
  zeros = jnp.zeros((LANES,), jnp.float32)

  row_vecs = [lane + g * LANES for g in range(GROUPS)]

  ids_vs = (ids_v0, ids_v1)
  counts_vs = (counts_v0, counts_v1)
  sems_in = (sem_in0, sem_in1)
  sems_out = (sem_out0, sem_out1)

  LU = 8

  def cp_in(ci, s):
    row0 = wid * ROWS_PER_W + ci * CHUNK
    return pltpu.make_async_copy(
        ids_hbm.at[pl.ds(row0, CHUNK)], ids_vs[s], sems_in[s])

  def cps_out(ci, s):
    row0 = wid * ROWS_PER_W + ci * CHUNK
    return (
        pltpu.make_async_copy(
            counts_vs[s].at[0], lo_hbm.at[pl.ds(row0, CHUNK)],
            sems_out[s]),
        pltpu.make_async_copy(
            counts_vs[s].at[1], hi_hbm.at[pl.ds(row0, CHUNK)],
            sems_out[s]),
    )

  def histogram(ids_v, counts_v):
    def zero_body(i, c):
      for half in range(2):
        for k in range(D // LANES):
          counts_v[half, i, pl.ds(k * LANES, LANES)] = zeros
      return c

    lax.fori_loop(0, CHUNK, zero_body, 0)

    def do_p(p_scalar, wrap):
      cols = lane + p_scalar
      if wrap:
        cols = jnp.where(cols >= PV, cols - PV, cols)
      vecs = [plsc.load_gather(ids_v, [row_vecs[g], cols])
              for g in range(GROUPS)]
      for g in range(GROUPS):
        w = vecs[g]
        hi = w >> 16
        plsc.addupdate_scatter(
            counts_v, [(w >> 7) & 1, row_vecs[g], w & 127], ones)
        plsc.addupdate_scatter(
            counts_v, [(hi >> 7) & 1, row_vecs[g], hi & 127], ones)

    PMAIN = PV - LANES
    PU = 4

    def p_body(po, c):
      for u in range(PU):
        do_p(po * PU + u, wrap=False)
      return c

    lax.fori_loop(0, PMAIN // PU, p_body, 0)

    def p_tail(po, c):
      for u in range(PU):
        do_p(PMAIN + po * PU + u, wrap=True)
      return c

    lax.fori_loop(0, (PV - PMAIN) // PU, p_tail, 0)

  cp_in(0, 0).start()
  cp_in(1, 1).start()

  def pair_body(p, carry):
    for k in range(2):
      ci = 2 * p + k
      s = k

      @pl.when(p > 0)
      def _():
        for h in cps_out(ci - 2, s):
          h.wait()

      cp_in(ci, s).wait()
      histogram(ids_vs[s], counts_vs[s])

      @pl.when(p < N_CHUNKS // 2 - 1)
      def _():
        cp_in(ci + 2, s).start()

      for h in cps_out(ci, s):
        h.start()
    return carry

  lax.fori_loop(0, N_CHUNKS // 2, pair_body, 0)
  for h in cps_out(N_CHUNKS - 2, 0):
    h.wait()
  for h in cps_out(N_CHUNKS - 1, 1):
    h.wait()


_hist = pl.kernel(
    _hist_body,
    out_type=(
        jax.ShapeDtypeStruct((B, D), jnp.float32),
        jax.ShapeDtypeStruct((B, D), jnp.float32),
    ),
    mesh=plsc.VectorSubcoreMesh(core_axis_name="c", subcore_axis_name="s",
                                num_cores=NUM_CORES, num_subcores=NUM_SUBCORES),
    scratch_types=[
        pltpu.VMEM((CHUNK, PP), jnp.int32),
        pltpu.VMEM((CHUNK, PP), jnp.int32),
        pltpu.VMEM((2, CHUNK, D), jnp.float32),
        pltpu.VMEM((2, CHUNK, D), jnp.float32),
        pltpu.SemaphoreType.DMA,
        pltpu.SemaphoreType.DMA,
        pltpu.SemaphoreType.DMA,
        pltpu.SemaphoreType.DMA,
    ],
    compiler_params=pltpu.CompilerParams(needs_layout_passes=False),
)

BR = 4096


def _mlp_body(lo_ref, hi_ref, e_ref, wt_ref, b_ref, out_ref):
  e = e_ref[...]
  pooled = (jnp.dot(lo_ref[...], e[:D, :], preferred_element_type=jnp.float32)
            + jnp.dot(hi_ref[...], e[D:, :],
                      preferred_element_type=jnp.float32)) * (1.0 / L)
  y = jnp.dot(pooled, wt_ref[...], preferred_element_type=jnp.float32)
  y = y + b_ref[...]
  out_ref[...] = 0.5 * y * (1.0 + lax.erf(y * (2.0 ** -0.5)))


_mlp = pl.pallas_call(
    _mlp_body,
    grid=(B // BR,),
    in_specs=[
        pl.BlockSpec((BR, D), lambda i: (i, 0)),
        pl.BlockSpec((BR, D), lambda i: (i, 0)),
        pl.BlockSpec((V, D), lambda i: (0, 0)),
        pl.BlockSpec((D, D), lambda i: (0, 0)),
        pl.BlockSpec((1, D), lambda i: (0, 0)),
    ],
    out_specs=pl.BlockSpec((BR, D), lambda i: (i, 0)),
    out_shape=jax.ShapeDtypeStruct((B, D), jnp.float32),
)


def kernel(ids, char_embed, W, b):
  pairs = lax.bitcast_convert_type(
      ids.astype(jnp.int16).reshape(B, PV, 2), jnp.int32)
  pairs = jnp.pad(pairs, ((0, 0), (0, PP - PV)))
  counts_lo, counts_hi = _hist(pairs)
  return _mlp(counts_lo, counts_hi, char_embed, W.T, b.reshape(1, D))


# speedup vs baseline: 2.2581x; 2.2581x over previous
import jax
import jax.numpy as jnp
from jax import lax
from jax.experimental import pallas as pl
from jax.experimental.pallas import tpu as pltpu
from jax.experimental.pallas import tpu_sc as plsc

B = 16384
L = 200
V = 256
D = 128

NUM_CORES = 2
NUM_SUBCORES = 16
NW = NUM_CORES * NUM_SUBCORES
LANES = 16

ROWS_PER_W = B // NW
CHUNK = 64
N_CHUNKS = ROWS_PER_W // CHUNK
GROUPS = CHUNK // LANES


def _hist_body(ids_hbm, lo_hbm, hi_hbm, ids_v0, ids_v1, counts_v0, counts_v1,
               sem_in0, sem_in1, sem_out0, sem_out1):
  wid = lax.axis_index("s") * NUM_CORES + lax.axis_index("c")
  lane = lax.iota(jnp.int32, LANES)
  ones = jnp.ones((LANES,), jnp.float32)
  zeros = jnp.zeros((LANES,), jnp.float32)

  row_vecs = [lane + g * LANES for g in range(GROUPS)]

  ids_vs = (ids_v0, ids_v1)
  counts_vs = (counts_v0, counts_v1)
  sems_in = (sem_in0, sem_in1)
  sems_out = (sem_out0, sem_out1)

  LU = 8

  def cp_in(ci, s):
    row0 = wid * ROWS_PER_W + ci * CHUNK
    return pltpu.make_async_copy(
        ids_hbm.at[pl.ds(row0, CHUNK)], ids_vs[s], sems_in[s])

  def cps_out(ci, s):
    row0 = wid * ROWS_PER_W + ci * CHUNK
    return (
        pltpu.make_async_copy(
            counts_vs[s].at[0], lo_hbm.at[pl.ds(row0, CHUNK)],
            sems_out[s]),
        pltpu.make_async_copy(
            counts_vs[s].at[1], hi_hbm.at[pl.ds(row0, CHUNK)],
            sems_out[s]),
    )

  def histogram(ids_v, counts_v):
    def zero_body(i, c):
      for half in range(2):
        for k in range(D // LANES):
          counts_v[half, i, pl.ds(k * LANES, LANES)] = zeros
      return c

    lax.fori_loop(0, CHUNK, zero_body, 0)

    def do_l(l_scalar, wrap):
      cols = lane + l_scalar
      if wrap:
        cols = jnp.where(cols >= L, cols - L, cols)
      vecs = [plsc.load_gather(ids_v, [row_vecs[g], cols])
              for g in range(GROUPS)]
      for g in range(GROUPS):
        idv = vecs[g]
        plsc.addupdate_scatter(
            counts_v, [idv >> 7, row_vecs[g], idv & 127], ones)

    LMAIN = L - LANES

    def l_body(lo, c):
      for u in range(LU):
        do_l(lo * LU + u, wrap=False)
      return c

    lax.fori_loop(0, LMAIN // LU, l_body, 0)

    def l_tail(lo, c):
      for u in range(LU):
        do_l(LMAIN + lo * LU + u, wrap=True)
      return c

    lax.fori_loop(0, (L - LMAIN) // LU, l_tail, 0)

  cp_in(0, 0).start()
  cp_in(1, 1).start()

  def pair_body(p, carry):
    for k in range(2):
      ci = 2 * p + k
      s = k

      @pl.when(p > 0)
      def _():
        for h in cps_out(ci - 2, s):
          h.wait()

      cp_in(ci, s).wait()
      histogram(ids_vs[s], counts_vs[s])

      @pl.when(p < N_CHUNKS // 2 - 1)
      def _():
        cp_in(ci + 2, s).start()

      for h in cps_out(ci, s):
        h.start()
    return carry

  lax.fori_loop(0, N_CHUNKS // 2, pair_body, 0)
  for h in cps_out(N_CHUNKS - 2, 0):
    h.wait()
  for h in cps_out(N_CHUNKS - 1, 1):
    h.wait()


_hist = pl.kernel(
    _hist_body,
    out_type=(
        jax.ShapeDtypeStruct((B, D), jnp.float32),
        jax.ShapeDtypeStruct((B, D), jnp.float32),
    ),
    mesh=plsc.VectorSubcoreMesh(core_axis_name="c", subcore_axis_name="s",
                                num_cores=NUM_CORES, num_subcores=NUM_SUBCORES),
    scratch_types=[
        pltpu.VMEM((CHUNK, L), jnp.int32),
        pltpu.VMEM((CHUNK, L), jnp.int32),
        pltpu.VMEM((2, CHUNK, D), jnp.float32),
        pltpu.VMEM((2, CHUNK, D), jnp.float32),
        pltpu.SemaphoreType.DMA,
        pltpu.SemaphoreType.DMA,
        pltpu.SemaphoreType.DMA,
        pltpu.SemaphoreType.DMA,
    ],
    compiler_params=pltpu.CompilerParams(needs_layout_passes=False),
)

BR = 4096


def _mlp_body(lo_ref, hi_ref, e_ref, wt_ref, b_ref, out_ref):
  e = e_ref[...]
  pooled = (jnp.dot(lo_ref[...], e[:D, :], preferred_element_type=jnp.float32)
            + jnp.dot(hi_ref[...], e[D:, :],
                      preferred_element_type=jnp.float32)) * (1.0 / L)
  y = jnp.dot(pooled, wt_ref[...], preferred_element_type=jnp.float32)
  y = y + b_ref[...]
  out_ref[...] = 0.5 * y * (1.0 + lax.erf(y * (2.0 ** -0.5)))


_mlp = pl.pallas_call(
    _mlp_body,
    grid=(B // BR,),
    in_specs=[
        pl.BlockSpec((BR, D), lambda i: (i, 0)),
        pl.BlockSpec((BR, D), lambda i: (i, 0)),
        pl.BlockSpec((V, D), lambda i: (0, 0)),
        pl.BlockSpec((D, D), lambda i: (0, 0)),
        pl.BlockSpec((1, D), lambda i: (0, 0)),
    ],
    out_specs=pl.BlockSpec((BR, D), lambda i: (i, 0)),
    out_shape=jax.ShapeDtypeStruct((B, D), jnp.float32),
)


def kernel(ids, char_embed, W, b):
  counts_lo, counts_hi = _hist(ids.astype(jnp.int32))
  return _mlp(counts_lo, counts_hi, char_embed, W.T, b.reshape(1, D))


# speedup vs baseline: 2.2975x; 1.0174x over previous
import jax
import jax.numpy as jnp
from jax import lax
from jax.experimental import pallas as pl
from jax.experimental.pallas import tpu as pltpu
from jax.experimental.pallas import tpu_sc as plsc

B = 16384
L = 200
V = 256
D = 128

NUM_CORES = 2
NUM_SUBCORES = 16
NW = NUM_CORES * NUM_SUBCORES
LANES = 16

ROWS_PER_W = B // NW
CHUNK = 64
N_CHUNKS = ROWS_PER_W // CHUNK
GROUPS = CHUNK // LANES


def _hist_body(ids_hbm, lo_hbm, hi_hbm, ids_v0, ids_v1, counts_v0, counts_v1,
               sem_in0, sem_in1, sem_out0, sem_out1):
  wid = lax.axis_index("s") * NUM_CORES + lax.axis_index("c")
  lane = lax.iota(jnp.int32, LANES)
  ones = jnp.ones((LANES,), jnp.float32)
  zeros = jnp.zeros((LANES,), jnp.float32)

  row_vecs = [lane + g * LANES for g in range(GROUPS)]

  ids_vs = (ids_v0, ids_v1)
  counts_vs = (counts_v0, counts_v1)
  sems_in = (sem_in0, sem_in1)
  sems_out = (sem_out0, sem_out1)

  LU = 8

  def cp_in(ci, s):
    row0 = wid * ROWS_PER_W + ci * CHUNK
    return pltpu.make_async_copy(
        ids_hbm.at[pl.ds(row0, CHUNK)], ids_vs[s], sems_in[s])

  def cps_out(ci, s):
    row0 = wid * ROWS_PER_W + ci * CHUNK
    return (
        pltpu.make_async_copy(
            counts_vs[s].at[0], lo_hbm.at[pl.ds(row0, CHUNK)],
            sems_out[s]),
        pltpu.make_async_copy(
            counts_vs[s].at[1], hi_hbm.at[pl.ds(row0, CHUNK)],
            sems_out[s]),
    )

  def histogram(ids_v, counts_v):
    def zero_body(i, c):
      for half in range(2):
        for k in range(D // LANES):
          counts_v[half, i, pl.ds(k * LANES, LANES)] = zeros
      return c

    lax.fori_loop(0, CHUNK, zero_body, 0)

    def do_l(l_scalar, wrap):
      cols = lane + l_scalar
      if wrap:
        cols = jnp.where(cols >= L, cols - L, cols)
      vecs = [plsc.load_gather(ids_v, [row_vecs[g], cols])
              for g in range(GROUPS)]
      for g in range(GROUPS):
        idv = vecs[g]
        plsc.addupdate_scatter(
            counts_v, [idv >> 7, row_vecs[g], idv & 127], ones)

    LMAIN = L - LANES

    def l_body(lo, c):
      for u in range(LU):
        do_l(lo * LU + u, wrap=False)
      return c

    lax.fori_loop(0, LMAIN // LU, l_body, 0)

    def l_tail(lo, c):
      for u in range(LU):
        do_l(LMAIN + lo * LU + u, wrap=True)
      return c

    lax.fori_loop(0, (L - LMAIN) // LU, l_tail, 0)

  cp_in(0, 0).start()
  cp_in(1, 1).start()

  def pair_body(p, carry):
    for k in range(2):
      ci = 2 * p + k
      s = k

      @pl.when(p > 0)
      def _():
        for h in cps_out(ci - 2, s):
          h.wait()

      cp_in(ci, s).wait()
      histogram(ids_vs[s], counts_vs[s])

      @pl.when(p < N_CHUNKS // 2 - 1)
      def _():
        cp_in(ci + 2, s).start()

      for h in cps_out(ci, s):
        h.start()
    return carry

  lax.fori_loop(0, N_CHUNKS // 2, pair_body, 0)
  for h in cps_out(N_CHUNKS - 2, 0):
    h.wait()
  for h in cps_out(N_CHUNKS - 1, 1):
    h.wait()


_hist = pl.kernel(
    _hist_body,
    out_type=(
        jax.ShapeDtypeStruct((B, D), jnp.float32),
        jax.ShapeDtypeStruct((B, D), jnp.float32),
    ),
    mesh=plsc.VectorSubcoreMesh(core_axis_name="c", subcore_axis_name="s",
                                num_cores=NUM_CORES, num_subcores=NUM_SUBCORES),
    scratch_types=[
        pltpu.VMEM((CHUNK, L), jnp.int32),
        pltpu.VMEM((CHUNK, L), jnp.int32),
        pltpu.VMEM((2, CHUNK, D), jnp.float32),
        pltpu.VMEM((2, CHUNK, D), jnp.float32),
        pltpu.SemaphoreType.DMA,
        pltpu.SemaphoreType.DMA,
        pltpu.SemaphoreType.DMA,
        pltpu.SemaphoreType.DMA,
    ],
    compiler_params=pltpu.CompilerParams(needs_layout_passes=False),
)

BR = 8192


def _mlp_body(lo_ref, hi_ref, e_ref, wt_ref, b_ref, out_ref):
  e = e_ref[...]
  pooled = (jnp.dot(lo_ref[...], e[:D, :], preferred_element_type=jnp.float32)
            + jnp.dot(hi_ref[...], e[D:, :],
                      preferred_element_type=jnp.float32)) * (1.0 / L)
  y = jnp.dot(pooled, wt_ref[...], preferred_element_type=jnp.float32)
  y = y + b_ref[...]
  out_ref[...] = 0.5 * y * (1.0 + lax.erf(y * (2.0 ** -0.5)))


_mlp = pl.pallas_call(
    _mlp_body,
    grid=(B // BR,),
    in_specs=[
        pl.BlockSpec((BR, D), lambda i: (i, 0)),
        pl.BlockSpec((BR, D), lambda i: (i, 0)),
        pl.BlockSpec((V, D), lambda i: (0, 0)),
        pl.BlockSpec((D, D), lambda i: (0, 0)),
        pl.BlockSpec((1, D), lambda i: (0, 0)),
    ],
    out_specs=pl.BlockSpec((BR, D), lambda i: (i, 0)),
    out_shape=jax.ShapeDtypeStruct((B, D), jnp.float32),
)


def kernel(ids, char_embed, W, b):
  counts_lo, counts_hi = _hist(ids.astype(jnp.int32))
  return _mlp(counts_lo, counts_hi, char_embed, W.T, b.reshape(1, D))
